# baseline (device time: 348960 ns/iter reference)
import jax
import jax.numpy as jnp
from jax import lax
from jax.experimental import pallas as pl
from jax.experimental.pallas import tpu as pltpu

N_EXP_LOCAL = 4
WIN = 640
OWN_WS = (0, 128, 384, 640)
PEER_WS = (0, 256, 512, 512)
RSTART = 896
REGION = 1152
XCHUNKS = ((896, 640), (1536, 512))


def kernel(x, assign, W1, W2):
    t, d = x.shape
    e_loc, _, f = W1.shape
    my_y = lax.axis_index("y")
    key = jnp.mod(assign - 4 * my_y, 8).astype(jnp.int32)
    perm = jnp.argsort(key)
    xs = jnp.take(x.astype(jnp.bfloat16), perm, axis=0)
    ks = jnp.take(key, perm, axis=0).reshape(t, 1)
    W1b = W1.astype(jnp.bfloat16)
    W2b = W2.astype(jnp.bfloat16)

    def body(x_ref, k_ref, w1_ref, w2_ref, out_ref,
             px_ref, pk_ref, sbuf_ref, rbuf_ref, w1b_ref, w2b_ref,
             xsends, xrecvs, asends, arecvs, psends, precvs, wsems):
        my_x = lax.axis_index("x")
        my_yy = lax.axis_index("y")
        my_z = lax.axis_index("z")
        peer = (my_x, 1 - my_yy, my_z)

        bsem = pltpu.get_barrier_semaphore()
        pl.semaphore_signal(bsem, inc=1, device_id=peer,
                            device_id_type=pl.DeviceIdType.MESH)
        pl.semaphore_wait(bsem, 1)

        rdma_k = pltpu.make_async_remote_copy(
            src_ref=k_ref.at[pl.ds(RSTART, REGION), :],
            dst_ref=pk_ref,
            send_sem=asends.at[0], recv_sem=arecvs.at[0],
            device_id=peer, device_id_type=pl.DeviceIdType.MESH)
        rdma_k.start()
        x_rdmas = []
        for ci, (g0, ln) in enumerate(XCHUNKS):
            r = pltpu.make_async_remote_copy(
                src_ref=x_ref.at[pl.ds(g0, ln), :],
                dst_ref=px_ref.at[pl.ds(g0 - RSTART, ln), :],
                send_sem=xsends.at[ci], recv_sem=xrecvs.at[ci],
                device_id=peer, device_id_type=pl.DeviceIdType.MESH)
            r.start()
            x_rdmas.append(r)

        out_ref[...] = jnp.zeros((t, d), jnp.float32)
        sbuf_ref[...] = jnp.zeros((REGION, d), jnp.bfloat16)

        def window(src_x_ref, src_k_ref, dst_ref, ws, mask_val, f32_dst):
            xc = src_x_ref[pl.ds(ws, WIN), :]
            kc = src_k_ref[pl.ds(ws, WIN), :]
            h = jnp.maximum(
                jnp.dot(xc, w1b_ref[0],
                        preferred_element_type=jnp.float32), 0.0)
            y = jnp.dot(h.astype(jnp.bfloat16), w2b_ref[0],
                        preferred_element_type=jnp.float32)
            contrib = jnp.where(kc == mask_val, y, 0.0)
            if not f32_dst:
                contrib = contrib.astype(jnp.bfloat16)
            dst_ref[pl.ds(ws, WIN), :] = dst_ref[pl.ds(ws, WIN), :] + contrib

        def own_window(e):
            window(x_ref, k_ref, out_ref, OWN_WS[e], e, f32_dst=True)

        def peer_window(e):
            window(px_ref, pk_ref, sbuf_ref, PEER_WS[e], 4 + e,
                   f32_dst=False)

        rdma_p = pltpu.make_async_remote_copy(
            src_ref=sbuf_ref, dst_ref=rbuf_ref,
            send_sem=psends.at[0], recv_sem=precvs.at[0],
            device_id=peer, device_id_type=pl.DeviceIdType.MESH)

        for e in range(N_EXP_LOCAL):
            d1 = pltpu.make_async_copy(w1_ref.at[e], w1b_ref.at[0],
                                       wsems.at[0])
            d2 = pltpu.make_async_copy(w2_ref.at[e], w2b_ref.at[0],
                                       wsems.at[1])
            d1.start()
            d2.start()
            d1.wait()
            d2.wait()
            if e == 0:
                own_window(0)
                rdma_k.wait_recv()
                x_rdmas[0].wait_recv()
                peer_window(0)
            elif e == 1:
                own_window(1)
                x_rdmas[1].wait_recv()
                peer_window(1)
            elif e == 2:
                own_window(2)
                peer_window(2)
            else:
                peer_window(3)
                rdma_p.start()
                own_window(3)

        rdma_p.wait_recv()
        out_ref[pl.ds(RSTART, REGION), :] = (
            out_ref[pl.ds(RSTART, REGION), :]
            + rbuf_ref[...].astype(jnp.float32))

        rdma_k.wait_send()
        for r in x_rdmas:
            r.wait_send()
        rdma_p.wait_send()

    out_sorted = pl.pallas_call(
        body,
        out_shape=jax.ShapeDtypeStruct((t, d), jnp.float32),
        in_specs=[
            pl.BlockSpec(memory_space=pltpu.VMEM),
            pl.BlockSpec(memory_space=pltpu.VMEM),
            pl.BlockSpec(memory_space=pl.ANY),
            pl.BlockSpec(memory_space=pl.ANY),
        ],
        out_specs=pl.BlockSpec(memory_space=pltpu.VMEM),
        scratch_shapes=[
            pltpu.VMEM((REGION, d), jnp.bfloat16),
            pltpu.VMEM((REGION, 1), jnp.int32),
            pltpu.VMEM((REGION, d), jnp.bfloat16),
            pltpu.VMEM((REGION, d), jnp.bfloat16),
            pltpu.VMEM((1, d, f), jnp.bfloat16),
            pltpu.VMEM((1, f, d), jnp.bfloat16),
            pltpu.SemaphoreType.DMA((2,)),
            pltpu.SemaphoreType.DMA((2,)),
            pltpu.SemaphoreType.DMA((1,)),
            pltpu.SemaphoreType.DMA((1,)),
            pltpu.SemaphoreType.DMA((1,)),
            pltpu.SemaphoreType.DMA((1,)),
            pltpu.SemaphoreType.DMA((2,)),
        ],
        compiler_params=pltpu.CompilerParams(collective_id=0),
    )(xs, ks, W1b, W2b)

    inv = jnp.zeros((t,), jnp.int32).at[perm].set(
        jnp.arange(t, dtype=jnp.int32))
    return jnp.take(out_sorted, inv, axis=0)


# device time: 154899 ns/iter; 2.2528x vs baseline; 2.2528x over previous
import jax
import jax.numpy as jnp
from jax import lax
from jax.experimental import pallas as pl
from jax.experimental.pallas import tpu as pltpu

N_EXP_LOCAL = 4
WIN = 512
OWN_WS = (0, 128, 384, 640)
PEER_WS = (0, 256, 512, 640)
RSTART = 896
REGION = 1152
XCHUNKS = ((896, 640), (1536, 512))
SORT_CHUNK = 512


def kernel(x, assign, W1, W2):
    t, d = x.shape
    e_loc, _, f = W1.shape
    my_y = lax.axis_index("y")
    kb = jnp.mod(assign - 4 * my_y, 8).astype(jnp.float32).reshape(t, 1)
    xb = x.astype(jnp.bfloat16)
    W1b = W1.astype(jnp.bfloat16)
    W2b = W2.astype(jnp.bfloat16)

    def body(x_ref, k_ref, w1_ref, w2_ref, out_ref,
             xs_ref, ks_ref, rank_ref, px_ref, pk_ref, sbuf_ref, rbuf_ref,
             w1b_ref, w2b_ref,
             xsends, xrecvs, asends, arecvs, psends, precvs, wsems):
        my_x = lax.axis_index("x")
        my_yy = lax.axis_index("y")
        my_z = lax.axis_index("z")
        peer = (my_x, 1 - my_yy, my_z)

        def wdma(e):
            d1 = pltpu.make_async_copy(w1_ref.at[e], w1b_ref.at[0],
                                       wsems.at[0])
            d2 = pltpu.make_async_copy(w2_ref.at[e], w2b_ref.at[0],
                                       wsems.at[1])
            d1.start()
            d2.start()
            return d1, d2

        wpend = wdma(0)

        keys = k_ref[...]
        lane128 = lax.broadcasted_iota(jnp.int32, (1, 128), 1).astype(jnp.float32)
        oh = jnp.where(keys == lane128, 1.0, 0.0).astype(jnp.bfloat16)
        lane_t = lax.broadcasted_iota(jnp.int32, (1, t), 1).astype(jnp.float32)
        cums = []
        for c0 in range(0, t, SORT_CHUNK):
            sub = c0 + lax.broadcasted_iota(
                jnp.int32, (SORT_CHUNK, 1), 0).astype(jnp.float32)
            l_c = jnp.where(lane_t <= sub, 1.0, 0.0).astype(jnp.bfloat16)
            cums.append(jnp.dot(l_c, oh, preferred_element_type=jnp.float32))
        cum = jnp.concatenate(cums, axis=0)
        totals = cum[t - 1:t, :]
        sub128 = lax.broadcasted_iota(jnp.int32, (128, 1), 0).astype(jnp.float32)
        strict_lower = jnp.where(sub128 < lane128, 1.0, 0.0).astype(
            jnp.bfloat16)
        tot_hi = jnp.floor(totals * (1.0 / 256.0))
        tot_lo = totals - 256.0 * tot_hi
        off = (jnp.dot(tot_hi.astype(jnp.bfloat16), strict_lower,
                       preferred_element_type=jnp.float32) * 256.0
               + jnp.dot(tot_lo.astype(jnp.bfloat16), strict_lower,
                         preferred_element_type=jnp.float32))
        rank_ref[...] = jnp.sum(
            oh.astype(jnp.float32) * (cum + off - 1.0),
            axis=1, keepdims=True)

        rank_all = rank_ref[...]
        for c0 in range(0, t, SORT_CHUNK):
            cols = c0 + lax.broadcasted_iota(
                jnp.int32, (1, SORT_CHUNK), 1).astype(jnp.float32)
            pt_c = jnp.where(rank_all == cols, 1.0, 0.0).astype(
                jnp.bfloat16)
            xs_ref[pl.ds(c0, SORT_CHUNK), :] = lax.dot_general(
                pt_c, x_ref[...], (((0,), (0,)), ((), ())),
                preferred_element_type=jnp.float32).astype(jnp.bfloat16)
            ks_ref[pl.ds(c0, SORT_CHUNK), :] = lax.dot_general(
                pt_c, keys.astype(jnp.bfloat16), (((0,), (0,)), ((), ())),
                preferred_element_type=jnp.float32).astype(jnp.bfloat16)

        bsem = pltpu.get_barrier_semaphore()
        pl.semaphore_signal(bsem, inc=1, device_id=peer,
                            device_id_type=pl.DeviceIdType.MESH)
        pl.semaphore_wait(bsem, 1)

        rdma_k = pltpu.make_async_remote_copy(
            src_ref=ks_ref.at[pl.ds(RSTART, REGION), :],
            dst_ref=pk_ref,
            send_sem=asends.at[0], recv_sem=arecvs.at[0],
            device_id=peer, device_id_type=pl.DeviceIdType.MESH)
        rdma_k.start()
        x_rdmas = []
        for ci, (g0, ln) in enumerate(XCHUNKS):
            r = pltpu.make_async_remote_copy(
                src_ref=xs_ref.at[pl.ds(g0, ln), :],
                dst_ref=px_ref.at[pl.ds(g0 - RSTART, ln), :],
                send_sem=xsends.at[ci], recv_sem=xrecvs.at[ci],
                device_id=peer, device_id_type=pl.DeviceIdType.MESH)
            r.start()
            x_rdmas.append(r)

        out_ref[...] = jnp.zeros((t, d), jnp.float32)
        sbuf_ref[...] = jnp.zeros((REGION, d), jnp.bfloat16)

        def window(src_x_ref, src_k_ref, dst_ref, ws, mask_val, f32_dst):
            xc = src_x_ref[pl.ds(ws, WIN), :]
            kc = src_k_ref[pl.ds(ws, WIN), :]
            h = jnp.maximum(
                jnp.dot(xc, w1b_ref[0],
                        preferred_element_type=jnp.float32), 0.0)
            y = jnp.dot(h.astype(jnp.bfloat16), w2b_ref[0],
                        preferred_element_type=jnp.float32)
            contrib = jnp.where(kc == float(mask_val), y, 0.0)
            if not f32_dst:
                contrib = contrib.astype(jnp.bfloat16)
            dst_ref[pl.ds(ws, WIN), :] = dst_ref[pl.ds(ws, WIN), :] + contrib

        def own_window(e):
            window(xs_ref, ks_ref, out_ref, OWN_WS[e], e, f32_dst=True)

        def peer_window(e):
            window(px_ref, pk_ref, sbuf_ref, PEER_WS[e], 4 + e,
                   f32_dst=False)

        rdma_p = pltpu.make_async_remote_copy(
            src_ref=sbuf_ref, dst_ref=rbuf_ref,
            send_sem=psends.at[0], recv_sem=precvs.at[0],
            device_id=peer, device_id_type=pl.DeviceIdType.MESH)

        for e in range(N_EXP_LOCAL):
            if e > 0:
                wpend = wdma(e)
            wpend[0].wait()
            wpend[1].wait()
            if e == 0:
                own_window(0)
                rdma_k.wait_recv()
                x_rdmas[0].wait_recv()
                peer_window(0)
            elif e == 1:
                own_window(1)
                x_rdmas[1].wait_recv()
                peer_window(1)
            elif e == 2:
                own_window(2)
                peer_window(2)
            else:
                peer_window(3)
                rdma_p.start()
                own_window(3)

        rdma_p.wait_recv()
        out_ref[pl.ds(RSTART, REGION), :] = (
            out_ref[pl.ds(RSTART, REGION), :]
            + rbuf_ref[...].astype(jnp.float32))

        ob = out_ref[...].astype(jnp.bfloat16)
        chunks = []
        for c0 in range(0, t, SORT_CHUNK):
            rc = rank_ref[pl.ds(c0, SORT_CHUNK), :]
            pt_r = jnp.where(rc == lane_t, 1.0, 0.0).astype(jnp.bfloat16)
            chunks.append(jnp.dot(pt_r, ob,
                                  preferred_element_type=jnp.float32))
        for i, c0 in enumerate(range(0, t, SORT_CHUNK)):
            out_ref[pl.ds(c0, SORT_CHUNK), :] = chunks[i]

        rdma_k.wait_send()
        for r in x_rdmas:
            r.wait_send()
        rdma_p.wait_send()

    return pl.pallas_call(
        body,
        out_shape=jax.ShapeDtypeStruct((t, d), jnp.float32),
        in_specs=[
            pl.BlockSpec(memory_space=pltpu.VMEM),
            pl.BlockSpec(memory_space=pltpu.VMEM),
            pl.BlockSpec(memory_space=pl.ANY),
            pl.BlockSpec(memory_space=pl.ANY),
        ],
        out_specs=pl.BlockSpec(memory_space=pltpu.VMEM),
        scratch_shapes=[
            pltpu.VMEM((t, d), jnp.bfloat16),
            pltpu.VMEM((t, 1), jnp.bfloat16),
            pltpu.VMEM((t, 1), jnp.float32),
            pltpu.VMEM((REGION, d), jnp.bfloat16),
            pltpu.VMEM((REGION, 1), jnp.bfloat16),
            pltpu.VMEM((REGION, d), jnp.bfloat16),
            pltpu.VMEM((REGION, d), jnp.bfloat16),
            pltpu.VMEM((1, d, f), jnp.bfloat16),
            pltpu.VMEM((1, f, d), jnp.bfloat16),
            pltpu.SemaphoreType.DMA((2,)),
            pltpu.SemaphoreType.DMA((2,)),
            pltpu.SemaphoreType.DMA((1,)),
            pltpu.SemaphoreType.DMA((1,)),
            pltpu.SemaphoreType.DMA((1,)),
            pltpu.SemaphoreType.DMA((1,)),
            pltpu.SemaphoreType.DMA((2,)),
        ],
        compiler_params=pltpu.CompilerParams(collective_id=0),
    )(xb, kb, W1b, W2b)
